# baseline (device time: 59857 ns/iter reference)
import jax
import jax.numpy as jnp
from jax import lax
from jax.experimental import pallas as pl
from jax.experimental.pallas import tpu as pltpu

N_GLOBAL = 4096
EPS = 1e-5
BM = 768


def _body(x_ref, gamma_ref, beta_ref, out_ref,
          send_buf, recv_buf, send_sems, recv_sems):
    i = pl.program_id(0)
    slot = lax.rem(i, 2)
    my_x = lax.axis_index("x")
    my_y = lax.axis_index("y")
    nbr = (my_x, 1 - my_y)

    @pl.when(i == 0)
    def _():
        barrier_sem = pltpu.get_barrier_semaphore()
        pl.semaphore_signal(barrier_sem, inc=1, device_id=nbr,
                            device_id_type=pl.DeviceIdType.MESH)
        pl.semaphore_wait(barrier_sem, 1)

    xb = x_ref[...]
    send_buf[slot, 0, :] = jnp.sum(xb, axis=1)
    send_buf[slot, 1, :] = jnp.sum(xb * xb, axis=1)

    rdma = pltpu.make_async_remote_copy(
        src_ref=send_buf.at[slot],
        dst_ref=recv_buf.at[slot],
        send_sem=send_sems.at[slot],
        recv_sem=recv_sems.at[slot],
        device_id=nbr,
        device_id_type=pl.DeviceIdType.MESH,
    )
    rdma.start()
    rdma.wait()

    tot = send_buf[slot] + recv_buf[slot]
    mean = tot[0, :] * (1.0 / N_GLOBAL)
    var = tot[1, :] * (1.0 / N_GLOBAL) - mean * mean
    rstd = lax.rsqrt(var + EPS)
    g = gamma_ref[...][None, :]
    b = beta_ref[...][None, :]
    out_ref[...] = (
        (xb - mean[:, None]) * rstd[:, None] * g + b
    ).astype(out_ref.dtype)


def kernel(x, gamma, beta):
    m, n = x.shape
    return pl.pallas_call(
        _body,
        grid=(m // BM,),
        in_specs=[
            pl.BlockSpec((BM, n), lambda i: (i, 0)),
            pl.BlockSpec((n,), lambda i: (0,)),
            pl.BlockSpec((n,), lambda i: (0,)),
        ],
        out_specs=pl.BlockSpec((BM, n), lambda i: (i, 0)),
        out_shape=jax.ShapeDtypeStruct((m, n), jnp.bfloat16),
        scratch_shapes=[
            pltpu.VMEM((2, 2, BM), jnp.float32),
            pltpu.VMEM((2, 2, BM), jnp.float32),
            pltpu.SemaphoreType.DMA((2,)),
            pltpu.SemaphoreType.DMA((2,)),
        ],
        compiler_params=pltpu.CompilerParams(
            dimension_semantics=("arbitrary",),
            collective_id=0,
        ),
    )(x, gamma, beta)


# device time: 53178 ns/iter; 1.1256x vs baseline; 1.1256x over previous
import jax
import jax.numpy as jnp
from jax import lax
from jax.experimental import pallas as pl
from jax.experimental.pallas import tpu as pltpu

N_GLOBAL = 4096
EPS = 1e-5
BM = 1536


def _body(x_ref, gamma_ref, beta_ref, out_ref,
          send_buf, recv_buf, send_sems, recv_sems):
    i = pl.program_id(0)
    slot = lax.rem(i, 2)
    my_x = lax.axis_index("x")
    my_y = lax.axis_index("y")
    nbr = (my_x, 1 - my_y)

    @pl.when(i == 0)
    def _():
        barrier_sem = pltpu.get_barrier_semaphore()
        pl.semaphore_signal(barrier_sem, inc=1, device_id=nbr,
                            device_id_type=pl.DeviceIdType.MESH)
        pl.semaphore_wait(barrier_sem, 1)

    xb = x_ref[...]
    send_buf[slot, 0, :] = jnp.sum(xb, axis=1)
    send_buf[slot, 1, :] = jnp.sum(xb * xb, axis=1)

    rdma = pltpu.make_async_remote_copy(
        src_ref=send_buf.at[slot],
        dst_ref=recv_buf.at[slot],
        send_sem=send_sems.at[slot],
        recv_sem=recv_sems.at[slot],
        device_id=nbr,
        device_id_type=pl.DeviceIdType.MESH,
    )
    rdma.start()
    rdma.wait()

    tot = send_buf[slot] + recv_buf[slot]
    mean = tot[0, :] * (1.0 / N_GLOBAL)
    var = tot[1, :] * (1.0 / N_GLOBAL) - mean * mean
    rstd = lax.rsqrt(var + EPS)
    g = gamma_ref[...][None, :]
    b = beta_ref[...][None, :]
    out_ref[...] = (
        (xb - mean[:, None]) * rstd[:, None] * g + b
    ).astype(out_ref.dtype)


def kernel(x, gamma, beta):
    m, n = x.shape
    return pl.pallas_call(
        _body,
        grid=(m // BM,),
        in_specs=[
            pl.BlockSpec((BM, n), lambda i: (i, 0)),
            pl.BlockSpec((n,), lambda i: (0,)),
            pl.BlockSpec((n,), lambda i: (0,)),
        ],
        out_specs=pl.BlockSpec((BM, n), lambda i: (i, 0)),
        out_shape=jax.ShapeDtypeStruct((m, n), jnp.bfloat16),
        scratch_shapes=[
            pltpu.VMEM((2, 2, BM), jnp.float32),
            pltpu.VMEM((2, 2, BM), jnp.float32),
            pltpu.SemaphoreType.DMA((2,)),
            pltpu.SemaphoreType.DMA((2,)),
        ],
        compiler_params=pltpu.CompilerParams(
            dimension_semantics=("arbitrary",),
            collective_id=0,
            vmem_limit_bytes=100 * 1024 * 1024,
        ),
    )(x, gamma, beta)


# device time: 47238 ns/iter; 1.2671x vs baseline; 1.1257x over previous
import jax
import jax.numpy as jnp
from jax import lax
from jax.experimental import pallas as pl
from jax.experimental.pallas import tpu as pltpu

N_GLOBAL = 4096
EPS = 1e-5
BM = 768
NBLK = 8
NXBUF = 4


def _body(x_hbm, gamma_ref, beta_ref, out_hbm,
          xbuf, obuf, send_buf, recv_buf,
          in_sems, out_sems, send_sems, recv_sems):
    my_x = lax.axis_index("x")
    my_y = lax.axis_index("y")
    nbr = (my_x, 1 - my_y)

    def in_copy(i):
        return pltpu.make_async_copy(
            x_hbm.at[pl.ds(i * BM, BM), :], xbuf.at[i % NXBUF],
            in_sems.at[i % NXBUF])

    def out_copy(i):
        return pltpu.make_async_copy(
            obuf.at[i % 2], out_hbm.at[pl.ds(i * BM, BM), :],
            out_sems.at[i % 2])

    def rdma(i):
        return pltpu.make_async_remote_copy(
            src_ref=send_buf.at[i],
            dst_ref=recv_buf.at[i],
            send_sem=send_sems.at[i],
            recv_sem=recv_sems.at[i],
            device_id=nbr,
            device_id_type=pl.DeviceIdType.MESH,
        )

    barrier_sem = pltpu.get_barrier_semaphore()
    pl.semaphore_signal(barrier_sem, inc=1, device_id=nbr,
                        device_id_type=pl.DeviceIdType.MESH)
    pl.semaphore_wait(barrier_sem, 1)

    in_copy(0).start()
    in_copy(1).start()

    g = gamma_ref[...][None, :]
    b = beta_ref[...][None, :]

    def normalize(i):
        rdma(i).wait_recv()
        tot = send_buf[i] + recv_buf[i]
        mean = tot[0, :] * (1.0 / N_GLOBAL)
        var = tot[1, :] * (1.0 / N_GLOBAL) - mean * mean
        rstd = lax.rsqrt(var + EPS)
        xb = xbuf[i % NXBUF]
        if i >= 2:
            out_copy(i - 2).wait()
        obuf[i % 2] = (
            (xb - mean[:, None]) * rstd[:, None] * g + b
        ).astype(obuf.dtype)
        out_copy(i).start()

    for i in range(NBLK):
        in_copy(i).wait()
        if i + 2 < NBLK:
            in_copy(i + 2).start()
        xb = xbuf[i % NXBUF]
        send_buf[i, 0, :] = jnp.sum(xb, axis=1)
        send_buf[i, 1, :] = jnp.sum(xb * xb, axis=1)
        rdma(i).start()
        if i >= 1:
            normalize(i - 1)

    normalize(NBLK - 1)
    for i in range(NBLK):
        rdma(i).wait_send()
    out_copy(NBLK - 2).wait()
    out_copy(NBLK - 1).wait()


def kernel(x, gamma, beta):
    m, n = x.shape
    return pl.pallas_call(
        _body,
        in_specs=[
            pl.BlockSpec(memory_space=pltpu.MemorySpace.HBM),
            pl.BlockSpec(memory_space=pltpu.MemorySpace.VMEM),
            pl.BlockSpec(memory_space=pltpu.MemorySpace.VMEM),
        ],
        out_specs=pl.BlockSpec(memory_space=pltpu.MemorySpace.HBM),
        out_shape=jax.ShapeDtypeStruct((m, n), jnp.bfloat16),
        scratch_shapes=[
            pltpu.VMEM((NXBUF, BM, n), jnp.float32),
            pltpu.VMEM((2, BM, n), jnp.bfloat16),
            pltpu.VMEM((NBLK, 2, BM), jnp.float32),
            pltpu.VMEM((NBLK, 2, BM), jnp.float32),
            pltpu.SemaphoreType.DMA((NXBUF,)),
            pltpu.SemaphoreType.DMA((2,)),
            pltpu.SemaphoreType.DMA((NBLK,)),
            pltpu.SemaphoreType.DMA((NBLK,)),
        ],
        compiler_params=pltpu.CompilerParams(
            collective_id=0,
            vmem_limit_bytes=100 * 1024 * 1024,
        ),
    )(x, gamma, beta)


# device time: 44709 ns/iter; 1.3388x vs baseline; 1.0566x over previous
import jax
import jax.numpy as jnp
from jax import lax
from jax.experimental import pallas as pl
from jax.experimental.pallas import tpu as pltpu

N_GLOBAL = 4096
EPS = 1e-5
BM = 768
NBLK = 8
NXBUF = 4


def _body(x_hbm, gamma_ref, beta_ref, out_hbm,
          xbuf, obuf, send_buf, recv_buf,
          in_sems, out_sems, send_sems, recv_sems):
    my_x = lax.axis_index("x")
    my_y = lax.axis_index("y")
    nbr = (my_x, 1 - my_y)

    def in_copy(i):
        return pltpu.make_async_copy(
            x_hbm.at[pl.ds(i * BM, BM), :], xbuf.at[i % NXBUF],
            in_sems.at[i % NXBUF])

    def out_copy(i):
        return pltpu.make_async_copy(
            obuf.at[i % 2], out_hbm.at[pl.ds(i * BM, BM), :],
            out_sems.at[i % 2])

    def rdma(i):
        return pltpu.make_async_remote_copy(
            src_ref=send_buf.at[i],
            dst_ref=recv_buf.at[i],
            send_sem=send_sems.at[i],
            recv_sem=recv_sems.at[i],
            device_id=nbr,
            device_id_type=pl.DeviceIdType.MESH,
        )

    barrier_sem = pltpu.get_barrier_semaphore()
    pl.semaphore_signal(barrier_sem, inc=1, device_id=nbr,
                        device_id_type=pl.DeviceIdType.MESH)
    pl.semaphore_wait(barrier_sem, 1)

    in_copy(0).start()
    in_copy(1).start()

    g = gamma_ref[...][None, :]
    b = beta_ref[...][None, :]

    def normalize(i):
        rdma(i).wait_recv()
        tot = send_buf[i] + recv_buf[i]
        mean = tot[0, :] * (1.0 / N_GLOBAL)
        var = tot[1, :] * (1.0 / N_GLOBAL) - mean * mean
        rstd = lax.rsqrt(var + EPS)
        mr = mean * rstd
        xb = xbuf[i % NXBUF]
        if i >= 2:
            out_copy(i - 2).wait()
        obuf[i % 2] = (
            (xb * rstd[:, None] - mr[:, None]) * g + b
        ).astype(obuf.dtype)
        out_copy(i).start()

    for i in range(NBLK):
        in_copy(i).wait()
        if i + 2 < NBLK:
            in_copy(i + 2).start()
        xb = xbuf[i % NXBUF]
        send_buf[i, 0, :] = jnp.sum(xb, axis=1)
        send_buf[i, 1, :] = jnp.sum(xb * xb, axis=1)
        rdma(i).start()
        if i >= 1:
            normalize(i - 1)

    normalize(NBLK - 1)
    for i in range(NBLK):
        rdma(i).wait_send()
    out_copy(NBLK - 2).wait()
    out_copy(NBLK - 1).wait()


def kernel(x, gamma, beta):
    m, n = x.shape
    return pl.pallas_call(
        _body,
        in_specs=[
            pl.BlockSpec(memory_space=pltpu.MemorySpace.HBM),
            pl.BlockSpec(memory_space=pltpu.MemorySpace.VMEM),
            pl.BlockSpec(memory_space=pltpu.MemorySpace.VMEM),
        ],
        out_specs=pl.BlockSpec(memory_space=pltpu.MemorySpace.HBM),
        out_shape=jax.ShapeDtypeStruct((m, n), jnp.bfloat16),
        scratch_shapes=[
            pltpu.VMEM((NXBUF, BM, n), jnp.float32),
            pltpu.VMEM((2, BM, n), jnp.bfloat16),
            pltpu.VMEM((NBLK, 2, BM), jnp.float32),
            pltpu.VMEM((NBLK, 2, BM), jnp.float32),
            pltpu.SemaphoreType.DMA((NXBUF,)),
            pltpu.SemaphoreType.DMA((2,)),
            pltpu.SemaphoreType.DMA((NBLK,)),
            pltpu.SemaphoreType.DMA((NBLK,)),
        ],
        compiler_params=pltpu.CompilerParams(
            collective_id=0,
            vmem_limit_bytes=100 * 1024 * 1024,
        ),
    )(x, gamma, beta)
